# baseline (speedup 1.0000x reference)
import jax
import jax.numpy as jnp
from jax.experimental import pallas as pl

_B, _H, _W, _C = 1, 512, 512, 96
_Ho, _Wo = _H // 2, _W // 2
_HB = 8


def _unpool_kernel(pi_ref, inp_ref, out_ref):
    x = pi_ref[0].reshape(_HB, 2, _W, _C)
    a = x[:, 0]
    b = x[:, 1]
    col = jax.lax.broadcasted_iota(jnp.int32, (_HB, _W, _C), 1)
    even = (col % 2) == 0
    al = jnp.roll(a, -1, axis=1)
    ar = jnp.roll(a, 1, axis=1)
    bl = jnp.roll(b, -1, axis=1)
    br = jnp.roll(b, 1, axis=1)
    a_o = jnp.where(even, al, ar)
    b_o = jnp.where(even, bl, br)
    mx = jnp.maximum(jnp.maximum(a, a_o), jnp.maximum(b, b_o))
    m_a = a == mx
    m_b = b == mx
    m_a_r = ar == mx
    m_a_l = al == mx
    m_b_r = br == mx
    f_a = m_a & (even | ~m_a_r)
    any_a = m_a | (even & m_a_l) | (~even & m_a_r)
    f_b = m_b & ~any_a & (even | ~m_b_r)
    v = jnp.repeat(inp_ref[0], 2, axis=1)
    z = jnp.zeros_like(v)
    oa = jnp.where(f_a, v, z)
    ob = jnp.where(f_b, v, z)
    out_ref[0] = jnp.stack([oa, ob], axis=1).reshape(2 * _HB, _W, _C)


def kernel(pool_input, pool_output, inputs):
    del pool_output
    return pl.pallas_call(
        _unpool_kernel,
        grid=(_Ho // _HB,),
        in_specs=[
            pl.BlockSpec((1, 2 * _HB, _W, _C), lambda i: (0, i, 0, 0)),
            pl.BlockSpec((1, _HB, _Wo, _C), lambda i: (0, i, 0, 0)),
        ],
        out_specs=pl.BlockSpec((1, 2 * _HB, _W, _C), lambda i: (0, i, 0, 0)),
        out_shape=jax.ShapeDtypeStruct((_B, _H, _W, _C), inputs.dtype),
    )(pool_input, inputs)


# speedup vs baseline: 1.0004x; 1.0004x over previous
import jax
import jax.numpy as jnp
from jax.experimental import pallas as pl
from jax.experimental.pallas import tpu as pltpu

_B, _H, _W, _C = 1, 512, 512, 96
_Ho, _Wo = _H // 2, _W // 2
_HB = 8


def _unpool_kernel(pi_ref, inp_ref, out_ref):
    x = pi_ref[0].reshape(_HB, 2, _W, _C)
    a = x[:, 0]
    b = x[:, 1]
    col = jax.lax.broadcasted_iota(jnp.int32, (_HB, _W, _C), 1)
    even = (col % 2) == 0
    al = jnp.roll(a, -1, axis=1)
    ar = jnp.roll(a, 1, axis=1)
    bl = jnp.roll(b, -1, axis=1)
    br = jnp.roll(b, 1, axis=1)
    a_o = jnp.where(even, al, ar)
    b_o = jnp.where(even, bl, br)
    mx = jnp.maximum(jnp.maximum(a, a_o), jnp.maximum(b, b_o))
    m_a = a == mx
    m_b = b == mx
    m_a_r = ar == mx
    m_a_l = al == mx
    m_b_r = br == mx
    f_a = m_a & (even | ~m_a_r)
    any_a = m_a | (even & m_a_l) | (~even & m_a_r)
    f_b = m_b & ~any_a & (even | ~m_b_r)
    v = jnp.repeat(inp_ref[0], 2, axis=1)
    z = jnp.zeros_like(v)
    oa = jnp.where(f_a, v, z)
    ob = jnp.where(f_b, v, z)
    out_ref[0] = jnp.stack([oa, ob], axis=1).reshape(2 * _HB, _W, _C)


def kernel(pool_input, pool_output, inputs):
    del pool_output
    return pl.pallas_call(
        _unpool_kernel,
        grid=(_Ho // _HB,),
        in_specs=[
            pl.BlockSpec((1, 2 * _HB, _W, _C), lambda i: (0, i, 0, 0)),
            pl.BlockSpec((1, _HB, _Wo, _C), lambda i: (0, i, 0, 0)),
        ],
        out_specs=pl.BlockSpec((1, 2 * _HB, _W, _C), lambda i: (0, i, 0, 0)),
        out_shape=jax.ShapeDtypeStruct((_B, _H, _W, _C), inputs.dtype),
        compiler_params=pltpu.CompilerParams(
            dimension_semantics=("parallel",)),
    )(pool_input, inputs)


# speedup vs baseline: 1.0464x; 1.0459x over previous
import jax
import jax.numpy as jnp
from jax.experimental import pallas as pl
from jax.experimental.pallas import tpu as pltpu

_B, _H, _W, _C = 1, 512, 512, 96
_Ho, _Wo = _H // 2, _W // 2
_HB = 8


def _unpool_kernel(pi_ref, inp_ref, out_ref):
    x = pi_ref[0].reshape(_HB, 2, _W, _C)
    a = x[:, 0]
    b = x[:, 1]
    col = jax.lax.broadcasted_iota(jnp.int32, (_HB, _W, _C), 1)
    even = (col % 2) == 0
    al = jnp.roll(a, -1, axis=1)
    ar = jnp.roll(a, 1, axis=1)
    bl = jnp.roll(b, -1, axis=1)
    br = jnp.roll(b, 1, axis=1)
    a_o = jnp.where(even, al, ar)
    b_o = jnp.where(even, bl, br)
    mx = jnp.maximum(jnp.maximum(a, a_o), jnp.maximum(b, b_o))
    m_a = a == mx
    m_b = b == mx
    m_ao = a_o == mx
    m_bo = b_o == mx
    f_a = m_a & (even | ~m_ao)
    f_b = m_b & ~(m_a | m_ao) & (even | ~m_bo)
    v = jnp.repeat(inp_ref[0], 2, axis=1)
    z = jnp.zeros_like(v)
    oa = jnp.where(f_a, v, z)
    ob = jnp.where(f_b, v, z)
    out_ref[0] = jnp.stack([oa, ob], axis=1).reshape(2 * _HB, _W, _C)


def kernel(pool_input, pool_output, inputs):
    del pool_output
    return pl.pallas_call(
        _unpool_kernel,
        grid=(_Ho // _HB,),
        in_specs=[
            pl.BlockSpec((1, 2 * _HB, _W, _C), lambda i: (0, i, 0, 0)),
            pl.BlockSpec((1, _HB, _Wo, _C), lambda i: (0, i, 0, 0)),
        ],
        out_specs=pl.BlockSpec((1, 2 * _HB, _W, _C), lambda i: (0, i, 0, 0)),
        out_shape=jax.ShapeDtypeStruct((_B, _H, _W, _C), inputs.dtype),
        compiler_params=pltpu.CompilerParams(
            dimension_semantics=("parallel",)),
    )(pool_input, inputs)


# speedup vs baseline: 1.0893x; 1.0410x over previous
import jax
import jax.numpy as jnp
from jax.experimental import pallas as pl
from jax.experimental.pallas import tpu as pltpu

_B, _H, _W, _C = 1, 512, 512, 96
_Ho, _Wo = _H // 2, _W // 2
_HB = 8


def _unpool_kernel(pi_ref, inp_ref, out_ref):
    a = pi_ref[:, 0]
    b = pi_ref[:, 1]
    col = jax.lax.broadcasted_iota(jnp.int32, (_HB, _W, _C), 1)
    even = (col % 2) == 0
    al = jnp.roll(a, -1, axis=1)
    ar = jnp.roll(a, 1, axis=1)
    bl = jnp.roll(b, -1, axis=1)
    br = jnp.roll(b, 1, axis=1)
    a_o = jnp.where(even, al, ar)
    b_o = jnp.where(even, bl, br)
    mx = jnp.maximum(jnp.maximum(a, a_o), jnp.maximum(b, b_o))
    m_a = a == mx
    m_b = b == mx
    m_ao = a_o == mx
    m_bo = b_o == mx
    f_a = m_a & (even | ~m_ao)
    f_b = m_b & ~(m_a | m_ao) & (even | ~m_bo)
    v = jnp.repeat(inp_ref[...], 2, axis=1)
    z = jnp.zeros_like(v)
    out_ref[:, 0] = jnp.where(f_a, v, z)
    out_ref[:, 1] = jnp.where(f_b, v, z)


def kernel(pool_input, pool_output, inputs):
    del pool_output
    pi = pool_input.reshape(_Ho, 2, _W, _C)
    inp = inputs.reshape(_Ho, _Wo, _C)
    out = pl.pallas_call(
        _unpool_kernel,
        grid=(_Ho // _HB,),
        in_specs=[
            pl.BlockSpec((_HB, 2, _W, _C), lambda i: (i, 0, 0, 0)),
            pl.BlockSpec((_HB, _Wo, _C), lambda i: (i, 0, 0)),
        ],
        out_specs=pl.BlockSpec((_HB, 2, _W, _C), lambda i: (i, 0, 0, 0)),
        out_shape=jax.ShapeDtypeStruct((_Ho, 2, _W, _C), inputs.dtype),
        compiler_params=pltpu.CompilerParams(
            dimension_semantics=("parallel",)),
    )(pi, inp)
    return out.reshape(_B, _H, _W, _C)
